# Initial kernel scaffold; baseline (speedup 1.0000x reference)
import jax
import jax.numpy as jnp
from jax.experimental import pallas as pl


def kernel(x, edge_index, edge_width, vertex_mass):
    raise NotImplementedError("write your pallas kernel here")


# speedup vs baseline: 11.0978x; 11.0978x over previous
import functools

import jax
import jax.numpy as jnp
import numpy as np
from jax import lax
from jax.experimental import pallas as pl
from jax.experimental.pallas import tpu as pltpu
from jax.experimental.pallas import tpu_sc as plsc

_C = 16
_PC = 2 * _C
_CH = 125
_G = 40
_ZR = 136


def _probe_consts():
    freqs = jnp.power(2.0, jnp.linspace(0.0, _C // 2 - 0.5, _C)).reshape(1, -1)
    freqs = freqs.astype(jnp.float32)
    pk = jax.random.key(42)
    kx, ky, kz, kp, kf = jax.random.split(pk, 5)
    x_w = jax.random.uniform(kx, (1, 1), dtype=jnp.float32)
    y_w = jax.random.uniform(ky, (1, 1), dtype=jnp.float32)
    z_w = jax.random.uniform(kz, (1, 1), dtype=jnp.float32)
    x_w = x_w / (x_w + y_w + z_w)
    y_w = y_w / (x_w + y_w + z_w)
    z_w = z_w / (x_w + y_w + z_w)
    phase = jax.random.uniform(kp, (1, 1), dtype=jnp.float32) * 2.0 * np.pi
    freq_noise = jax.random.uniform(kf, (1, 1), dtype=jnp.float32) * 0.5 + 0.75
    freqs = freqs * freq_noise
    amp = 0.5 / freqs
    wvec = jnp.concatenate([x_w, y_w, z_w, phase], axis=1)
    return wvec, freqs, amp


def _probe_body(x_ref, w_ref, f_ref, a_ref, out_ref):
    xw = w_ref[0, 0]
    yw = w_ref[0, 1]
    zw = w_ref[0, 2]
    ph = w_ref[0, 3]
    cs = x_ref[:, 0:1] * xw + x_ref[:, 1:2] * yw + x_ref[:, 2:3] * zw
    pin = cs * f_ref[0:1, :] + ph
    s = a_ref[0:1, :] * jnp.sin(pin)
    c = a_ref[0:1, :] * jnp.cos(pin)
    out_ref[...] = jnp.concatenate([s, c], axis=1)


def _make_probe(n, br=2000):
    grid = n // br
    return pl.pallas_call(
        _probe_body,
        grid=(grid,),
        in_specs=[
            pl.BlockSpec((br, 3), lambda i: (i, 0)),
            pl.BlockSpec((1, 4), lambda i: (0, 0)),
            pl.BlockSpec((1, _C), lambda i: (0, 0)),
            pl.BlockSpec((1, _C), lambda i: (0, 0)),
        ],
        out_specs=pl.BlockSpec((br, _PC), lambda i: (i, 0)),
        out_shape=jax.ShapeDtypeStruct((n, _PC), jnp.float32),
    )


def _comb_body(a0_ref, a1_ref, m_ref, out_ref):
    out_ref[...] = (a0_ref[...] + a1_ref[...]) * (1.0 / m_ref[...])


def _make_combine(n, br=2000):
    grid = n // br
    return pl.pallas_call(
        _comb_body,
        grid=(grid,),
        in_specs=[
            pl.BlockSpec((br, _PC), lambda i: (i, 0)),
            pl.BlockSpec((br, _PC), lambda i: (i, 0)),
            pl.BlockSpec((br, 1), lambda i: (i, 0)),
        ],
        out_specs=pl.BlockSpec((br, _PC), lambda i: (i, 0)),
        out_shape=jax.ShapeDtypeStruct((n, _PC), jnp.float32),
    )


def _make_sc_scatter(n, e):
    info = plsc.get_sparse_core_info()
    nc, ns, nl = info.num_cores, info.num_subcores, info.num_lanes
    nw = nc * ns
    rows_total = e // _CH
    rows_per_w = rows_total // nw
    blocks = rows_per_w // _G
    npt = -(-(n // ns) // _ZR) * _ZR
    n_pad = npt * ns
    mesh = plsc.VectorSubcoreMesh(core_axis_name="c", subcore_axis_name="s")

    @functools.partial(
        pl.kernel,
        out_type=jax.ShapeDtypeStruct((nc, n_pad, _PC), jnp.float32),
        mesh=mesh,
        scratch_types=[
            pltpu.VMEM((_G, _CH), jnp.int32),
            pltpu.VMEM((_G, _CH), jnp.int32),
            pltpu.VMEM((_G * _CH + 16,), jnp.float32),
            pltpu.VMEM((_CH, _PC), jnp.float32),
            pltpu.VMEM((_CH, _PC), jnp.float32),
            pltpu.VMEM((_ZR, _PC), jnp.float32),
            pltpu.VMEM_SHARED((n_pad, _PC), jnp.float32),
            pltpu.SemaphoreType.DMA,
        ],
        compiler_params=pltpu.CompilerParams(use_tc_tiling_on_sc=False),
    )
    def sc_scatter(probe_hbm, src_hbm, dst_hbm, w_hbm, out_hbm,
                   sbuf, dbuf, wbuf, rows, negb, zbuf, acc, sem):
        cid = lax.axis_index("c")
        sid = lax.axis_index("s")
        wid = sid * nc + cid

        def _zfill(i, _):
            zbuf[i, pl.ds(0, nl)] = jnp.zeros((nl,), jnp.float32)
            zbuf[i, pl.ds(nl, nl)] = jnp.zeros((nl,), jnp.float32)
            return 0

        lax.fori_loop(0, _ZR, _zfill, 0)

        def _zcopy(t, _):
            pltpu.sync_copy(zbuf, acc.at[pl.ds(sid * npt + t * _ZR, _ZR)])
            return 0

        lax.fori_loop(0, npt // _ZR, _zcopy, 0)
        plsc.subcore_barrier()

        wbase = wid * rows_per_w

        def _block(t, _):
            off = wbase + t * _G
            pltpu.sync_copy(src_hbm.at[pl.ds(off, _G)], sbuf)
            pltpu.sync_copy(dst_hbm.at[pl.ds(off, _G)], dbuf)
            pltpu.sync_copy(w_hbm.at[pl.ds(off * _CH, _G * _CH)],
                            wbuf.at[pl.ds(0, _G * _CH)])

            def _chunk(k, _):
                pltpu.async_copy(probe_hbm.at[sbuf.at[k]], rows, sem).wait()

                def _scale(i, _):
                    wv = wbuf[pl.ds(k * _CH + i, nl)]
                    wb = jnp.full((nl,), wv[0], jnp.float32)
                    a = rows[i, pl.ds(0, nl)] * wb
                    b = rows[i, pl.ds(nl, nl)] * wb
                    rows[i, pl.ds(0, nl)] = a
                    rows[i, pl.ds(nl, nl)] = b
                    negb[i, pl.ds(0, nl)] = -a
                    negb[i, pl.ds(nl, nl)] = -b
                    return 0

                lax.fori_loop(0, _CH, _scale, 0)
                pltpu.sync_copy(rows, acc.at[dbuf.at[k]], add=True)
                pltpu.sync_copy(negb, acc.at[sbuf.at[k]], add=True)
                return 0

            lax.fori_loop(0, _G, _chunk, 0)
            return 0

        lax.fori_loop(0, blocks, _block, 0)
        plsc.subcore_barrier()

        pltpu.sync_copy(acc.at[pl.ds(sid * npt, npt)],
                        out_hbm.at[cid, pl.ds(sid * npt, npt)])

    return sc_scatter


def kernel(x, edge_index, edge_width, vertex_mass):
    n = x.shape[0]
    e = edge_index.shape[1]
    src = edge_index[0].astype(jnp.int32)
    dst = edge_index[1].astype(jnp.int32)
    w = edge_width.reshape(e).astype(jnp.float32)

    wvec, freqs, amp = _probe_consts()
    probe = _make_probe(n)(x, wvec, freqs, amp)

    rows_total = e // _CH
    src2 = src.reshape(rows_total, _CH)
    dst2 = dst.reshape(rows_total, _CH)
    accs = _make_sc_scatter(n, e)(probe, src2, dst2, w)

    out = _make_combine(n)(accs[0, :n], accs[1, :n], vertex_mass)
    return out


# speedup vs baseline: 12.9539x; 1.1672x over previous
import functools

import jax
import jax.numpy as jnp
import numpy as np
from jax import lax
from jax.experimental import pallas as pl
from jax.experimental.pallas import tpu as pltpu
from jax.experimental.pallas import tpu_sc as plsc

_C = 16
_PC = 2 * _C
_CH = 128
_G = 8
_BLK = 49
_ZR = 144


def _probe_consts():
    freqs = jnp.power(2.0, jnp.linspace(0.0, _C // 2 - 0.5, _C)).reshape(1, -1)
    freqs = freqs.astype(jnp.float32)
    pk = jax.random.key(42)
    kx, ky, kz, kp, kf = jax.random.split(pk, 5)
    x_w = jax.random.uniform(kx, (1, 1), dtype=jnp.float32)
    y_w = jax.random.uniform(ky, (1, 1), dtype=jnp.float32)
    z_w = jax.random.uniform(kz, (1, 1), dtype=jnp.float32)
    x_w = x_w / (x_w + y_w + z_w)
    y_w = y_w / (x_w + y_w + z_w)
    z_w = z_w / (x_w + y_w + z_w)
    phase = jax.random.uniform(kp, (1, 1), dtype=jnp.float32) * 2.0 * np.pi
    freq_noise = jax.random.uniform(kf, (1, 1), dtype=jnp.float32) * 0.5 + 0.75
    freqs = freqs * freq_noise
    amp = 0.5 / freqs
    wvec = jnp.concatenate([x_w, y_w, z_w, phase], axis=1)
    return wvec, freqs, amp


def _probe_body(x_ref, w_ref, f_ref, a_ref, out_ref):
    xw = w_ref[0, 0]
    yw = w_ref[0, 1]
    zw = w_ref[0, 2]
    ph = w_ref[0, 3]
    cs = x_ref[:, 0:1] * xw + x_ref[:, 1:2] * yw + x_ref[:, 2:3] * zw
    pin = cs * f_ref[0:1, :] + ph
    s = a_ref[0:1, :] * jnp.sin(pin)
    c = a_ref[0:1, :] * jnp.cos(pin)
    out_ref[...] = jnp.concatenate([s, c], axis=1)


def _make_probe(n, br=2000):
    grid = n // br
    return pl.pallas_call(
        _probe_body,
        grid=(grid,),
        in_specs=[
            pl.BlockSpec((br, 3), lambda i: (i, 0)),
            pl.BlockSpec((1, 4), lambda i: (0, 0)),
            pl.BlockSpec((1, _C), lambda i: (0, 0)),
            pl.BlockSpec((1, _C), lambda i: (0, 0)),
        ],
        out_specs=pl.BlockSpec((br, _PC), lambda i: (i, 0)),
        out_shape=jax.ShapeDtypeStruct((n, _PC), jnp.float32),
    )


def _comb_body(a0_ref, a1_ref, e0_ref, e1_ref, p_ref, m_ref, out_ref):
    ews = e0_ref[...] + e1_ref[...]
    acc = a0_ref[...] + a1_ref[...]
    out_ref[...] = (acc - ews * p_ref[...]) * (1.0 / m_ref[...])


def _make_combine(n, br=2000):
    grid = n // br
    return pl.pallas_call(
        _comb_body,
        grid=(grid,),
        in_specs=[
            pl.BlockSpec((br, _PC), lambda i: (i, 0)),
            pl.BlockSpec((br, _PC), lambda i: (i, 0)),
            pl.BlockSpec((br, 1), lambda i: (i, 0)),
            pl.BlockSpec((br, 1), lambda i: (i, 0)),
            pl.BlockSpec((br, _PC), lambda i: (i, 0)),
            pl.BlockSpec((br, 1), lambda i: (i, 0)),
        ],
        out_specs=pl.BlockSpec((br, _PC), lambda i: (i, 0)),
        out_shape=jax.ShapeDtypeStruct((n, _PC), jnp.float32),
    )


def _make_sc_scatter(n, rows_per_w):
    info = plsc.get_sparse_core_info()
    nc, ns, nl = info.num_cores, info.num_subcores, info.num_lanes
    nw = nc * ns
    rows_total = rows_per_w * nw
    npt = -(-(n // ns) // _ZR) * _ZR
    n_pad = npt * ns
    mesh = plsc.VectorSubcoreMesh(core_axis_name="c", subcore_axis_name="s")

    @functools.partial(
        pl.kernel,
        out_type=(jax.ShapeDtypeStruct((nc, n_pad, _PC), jnp.float32),
                  jax.ShapeDtypeStruct((nc, n_pad), jnp.float32)),
        mesh=mesh,
        scratch_types=[
            pltpu.VMEM((_G, _CH), jnp.int32),
            pltpu.VMEM((_G, _CH), jnp.int32),
            pltpu.VMEM((_G, _CH), jnp.float32),
            pltpu.VMEM((_CH, _PC), jnp.float32),
            pltpu.VMEM((_ZR, _PC), jnp.float32),
            pltpu.VMEM((_ZR,), jnp.float32),
            pltpu.VMEM_SHARED((n_pad, _PC), jnp.float32),
            pltpu.VMEM_SHARED((n_pad,), jnp.float32),
            pltpu.SemaphoreType.DMA,
        ],
        compiler_params=pltpu.CompilerParams(
            use_tc_tiling_on_sc=False, needs_layout_passes=False),
    )
    def sc_scatter(probe_hbm, src_hbm, dst_hbm, w_hbm, acc_hbm, ews_hbm,
                   sbuf, dbuf, wbuf, rows, zbuf, zbuf1, acc, ews, sem):
        cid = lax.axis_index("c")
        sid = lax.axis_index("s")
        wid = sid * nc + cid

        def _zfill(i, _):
            zbuf[i, pl.ds(0, nl)] = jnp.zeros((nl,), jnp.float32)
            zbuf[i, pl.ds(nl, nl)] = jnp.zeros((nl,), jnp.float32)
            return 0

        lax.fori_loop(0, _ZR, _zfill, 0)

        def _zfill1(i, _):
            zbuf1[pl.ds(i * nl, nl)] = jnp.zeros((nl,), jnp.float32)
            return 0

        lax.fori_loop(0, _ZR // nl, _zfill1, 0)

        def _zcopy(t, _):
            pltpu.sync_copy(zbuf, acc.at[pl.ds(sid * npt + t * _ZR, _ZR)])
            pltpu.sync_copy(zbuf1, ews.at[pl.ds(sid * npt + t * _ZR, _ZR)])
            return 0

        lax.fori_loop(0, npt // _ZR, _zcopy, 0)
        plsc.subcore_barrier()

        wbase = wid * rows_per_w

        def _block(t, _):
            off = wbase + t * _G
            pltpu.sync_copy(src_hbm.at[pl.ds(off, _G)], sbuf)
            pltpu.sync_copy(dst_hbm.at[pl.ds(off, _G)], dbuf)
            pltpu.sync_copy(w_hbm.at[pl.ds(off, _G)], wbuf)

            def _chunk(k, _):
                pltpu.async_copy(probe_hbm.at[sbuf.at[k]], rows, sem).wait()

                def _group(g, _):
                    wv = wbuf[k, pl.ds(g * nl, nl)]
                    for i in range(nl):
                        r = g * nl + i
                        wb = jnp.full((nl,), wv[i], jnp.float32)
                        rows[r, pl.ds(0, nl)] = rows[r, pl.ds(0, nl)] * wb
                        rows[r, pl.ds(nl, nl)] = rows[r, pl.ds(nl, nl)] * wb
                    return 0

                lax.fori_loop(0, _CH // nl, _group, 0)
                pltpu.sync_copy(rows, acc.at[dbuf.at[k]], add=True)
                pltpu.sync_copy(wbuf.at[k], ews.at[sbuf.at[k]], add=True)
                return 0

            lax.fori_loop(0, _G, _chunk, 0)
            return 0

        lax.fori_loop(0, _BLK, _block, 0)
        plsc.subcore_barrier()

        pltpu.sync_copy(acc.at[pl.ds(sid * npt, npt)],
                        acc_hbm.at[cid, pl.ds(sid * npt, npt)])
        pltpu.sync_copy(ews.at[pl.ds(sid * npt, npt)],
                        ews_hbm.at[cid, pl.ds(sid * npt, npt)])

    return sc_scatter


def kernel(x, edge_index, edge_width, vertex_mass):
    n = x.shape[0]
    e = edge_index.shape[1]
    src = edge_index[0].astype(jnp.int32)
    dst = edge_index[1].astype(jnp.int32)
    w = edge_width.reshape(e).astype(jnp.float32)

    wvec, freqs, amp = _probe_consts()
    probe = _make_probe(n)(x, wvec, freqs, amp)

    nw = 32
    epw = e // nw
    epw_pad = _CH * _G * _BLK
    pad = epw_pad - epw
    didx = jnp.broadcast_to(
        jnp.arange(pad, dtype=jnp.int32) % n, (nw, pad))
    src_p = jnp.concatenate(
        [src.reshape(nw, epw), didx], axis=1).reshape(-1, _CH)
    dst_p = jnp.concatenate(
        [dst.reshape(nw, epw), didx], axis=1).reshape(-1, _CH)
    w_p = jnp.concatenate(
        [w.reshape(nw, epw), jnp.zeros((nw, pad), jnp.float32)],
        axis=1).reshape(-1, _CH)

    accs, ews = _make_sc_scatter(n, epw_pad // _CH)(probe, src_p, dst_p, w_p)

    out = _make_combine(n)(
        accs[0, :n], accs[1, :n],
        ews[0, :n, None], ews[1, :n, None],
        probe, vertex_mass)
    return out


# speedup vs baseline: 17.0569x; 1.3167x over previous
import functools

import jax
import jax.numpy as jnp
import numpy as np
from jax import lax
from jax.experimental import pallas as pl
from jax.experimental.pallas import tpu as pltpu
from jax.experimental.pallas import tpu_sc as plsc

_C = 16
_PC = 2 * _C
_CH = 128
_G = 8
_BLK = 49
_ZR = 48
_NB = 4


def _probe_consts():
    freqs = jnp.power(2.0, jnp.linspace(0.0, _C // 2 - 0.5, _C)).reshape(1, -1)
    freqs = freqs.astype(jnp.float32)
    pk = jax.random.key(42)
    kx, ky, kz, kp, kf = jax.random.split(pk, 5)
    x_w = jax.random.uniform(kx, (1, 1), dtype=jnp.float32)
    y_w = jax.random.uniform(ky, (1, 1), dtype=jnp.float32)
    z_w = jax.random.uniform(kz, (1, 1), dtype=jnp.float32)
    x_w = x_w / (x_w + y_w + z_w)
    y_w = y_w / (x_w + y_w + z_w)
    z_w = z_w / (x_w + y_w + z_w)
    phase = jax.random.uniform(kp, (1, 1), dtype=jnp.float32) * 2.0 * np.pi
    freq_noise = jax.random.uniform(kf, (1, 1), dtype=jnp.float32) * 0.5 + 0.75
    freqs = freqs * freq_noise
    amp = 0.5 / freqs
    wvec = jnp.concatenate([x_w, y_w, z_w, phase], axis=1)
    return wvec, freqs, amp


def _probe_body(x_ref, w_ref, f_ref, a_ref, out_ref):
    xw = w_ref[0, 0]
    yw = w_ref[0, 1]
    zw = w_ref[0, 2]
    ph = w_ref[0, 3]
    cs = x_ref[:, 0:1] * xw + x_ref[:, 1:2] * yw + x_ref[:, 2:3] * zw
    pin = cs * f_ref[0:1, :] + ph
    s = a_ref[0:1, :] * jnp.sin(pin)
    c = a_ref[0:1, :] * jnp.cos(pin)
    out_ref[...] = jnp.concatenate([s, c], axis=1)


def _make_probe(n, br=2000):
    grid = n // br
    return pl.pallas_call(
        _probe_body,
        grid=(grid,),
        in_specs=[
            pl.BlockSpec((br, 3), lambda i: (i, 0)),
            pl.BlockSpec((1, 4), lambda i: (0, 0)),
            pl.BlockSpec((1, _C), lambda i: (0, 0)),
            pl.BlockSpec((1, _C), lambda i: (0, 0)),
        ],
        out_specs=pl.BlockSpec((br, _PC), lambda i: (i, 0)),
        out_shape=jax.ShapeDtypeStruct((n, _PC), jnp.float32),
    )


def _comb_body(a0_ref, a1_ref, e0_ref, e1_ref, p_ref, m_ref, out_ref):
    ews = e0_ref[...] + e1_ref[...]
    acc = a0_ref[...] + a1_ref[...]
    out_ref[...] = (acc - ews * p_ref[...]) * (1.0 / m_ref[...])


def _make_combine(n, br=2000):
    grid = n // br
    return pl.pallas_call(
        _comb_body,
        grid=(grid,),
        in_specs=[
            pl.BlockSpec((br, _PC), lambda i: (i, 0)),
            pl.BlockSpec((br, _PC), lambda i: (i, 0)),
            pl.BlockSpec((br, 1), lambda i: (i, 0)),
            pl.BlockSpec((br, 1), lambda i: (i, 0)),
            pl.BlockSpec((br, _PC), lambda i: (i, 0)),
            pl.BlockSpec((br, 1), lambda i: (i, 0)),
        ],
        out_specs=pl.BlockSpec((br, _PC), lambda i: (i, 0)),
        out_shape=jax.ShapeDtypeStruct((n, _PC), jnp.float32),
    )


def _make_sc_scatter(n, rows_per_w):
    info = plsc.get_sparse_core_info()
    nc, ns, nl = info.num_cores, info.num_subcores, info.num_lanes
    nw = nc * ns
    rows_total = rows_per_w * nw
    npt = -(-(n // ns) // _ZR) * _ZR
    n_pad = npt * ns
    mesh = plsc.VectorSubcoreMesh(core_axis_name="c", subcore_axis_name="s")

    @functools.partial(
        pl.kernel,
        out_type=(jax.ShapeDtypeStruct((nc, n_pad, _PC), jnp.float32),
                  jax.ShapeDtypeStruct((nc, n_pad), jnp.float32)),
        mesh=mesh,
        scratch_types=[
            pltpu.VMEM((_G, _CH), jnp.int32),
            pltpu.VMEM((_G, _CH), jnp.int32),
            pltpu.VMEM((_G, _CH), jnp.float32),
            [pltpu.VMEM((_CH, _PC), jnp.float32) for _ in range(_NB)],
            pltpu.VMEM((_ZR, _PC), jnp.float32),
            pltpu.VMEM((_ZR,), jnp.float32),
            pltpu.VMEM_SHARED((n_pad, _PC), jnp.float32),
            pltpu.VMEM_SHARED((n_pad,), jnp.float32),
            [pltpu.SemaphoreType.DMA for _ in range(_NB)],
            [pltpu.SemaphoreType.DMA for _ in range(_NB)],
            pltpu.SemaphoreType.DMA,
        ],
        compiler_params=pltpu.CompilerParams(
            use_tc_tiling_on_sc=False, needs_layout_passes=False),
    )
    def sc_scatter(probe_hbm, src_hbm, dst_hbm, w_hbm, acc_hbm, ews_hbm,
                   sbuf, dbuf, wbuf, rows, zbuf, zbuf1, acc, ews,
                   gsem, ssem, esem):
        cid = lax.axis_index("c")
        sid = lax.axis_index("s")
        wid = sid * nc + cid

        def _zfill(i, _):
            zbuf[i, pl.ds(0, nl)] = jnp.zeros((nl,), jnp.float32)
            zbuf[i, pl.ds(nl, nl)] = jnp.zeros((nl,), jnp.float32)
            return 0

        lax.fori_loop(0, _ZR, _zfill, 0)

        def _zfill1(i, _):
            zbuf1[pl.ds(i * nl, nl)] = jnp.zeros((nl,), jnp.float32)
            return 0

        lax.fori_loop(0, _ZR // nl, _zfill1, 0)

        def _zcopy(t, _):
            pltpu.sync_copy(zbuf, acc.at[pl.ds(sid * npt + t * _ZR, _ZR)])
            pltpu.sync_copy(zbuf1, ews.at[pl.ds(sid * npt + t * _ZR, _ZR)])
            return 0

        lax.fori_loop(0, npt // _ZR, _zcopy, 0)
        plsc.subcore_barrier()

        wbase = wid * rows_per_w

        def _scale(k, buf):
            def _group(g, _):
                wv = wbuf[k, pl.ds(g * nl, nl)]
                for i in range(nl):
                    r = g * nl + i
                    wb = jnp.full((nl,), wv[i], jnp.float32)
                    buf[r, pl.ds(0, nl)] = buf[r, pl.ds(0, nl)] * wb
                    buf[r, pl.ds(nl, nl)] = buf[r, pl.ds(nl, nl)] * wb
                return 0

            lax.fori_loop(0, _CH // nl, _group, 0)

        def _block(t, _):
            @pl.when(t > 0)
            def _():
                pltpu.make_async_copy(
                    rows[2], acc.at[dbuf.at[_G - 2]], ssem[2]).wait()
                pltpu.make_async_copy(
                    rows[3], acc.at[dbuf.at[_G - 1]], ssem[3]).wait()

            off = wbase + t * _G
            pltpu.sync_copy(src_hbm.at[pl.ds(off, _G)], sbuf)
            pltpu.sync_copy(dst_hbm.at[pl.ds(off, _G)], dbuf)
            pltpu.sync_copy(w_hbm.at[pl.ds(off, _G)], wbuf)

            edescs = []
            for u in range(_G):
                edescs.append(pltpu.async_copy(
                    wbuf.at[u], ews.at[sbuf.at[u]], esem, add=True))

            gdescs = {}
            sdescs = {}

            def _issue_g(u):
                gdescs[u] = pltpu.async_copy(
                    probe_hbm.at[sbuf.at[u]], rows[u % _NB], gsem[u % _NB])

            _issue_g(0)
            _issue_g(1)
            for u in range(_G):
                b = u % _NB
                gdescs[u].wait()
                _scale(u, rows[b])
                sdescs[u] = pltpu.async_copy(
                    rows[b], acc.at[dbuf.at[u]], ssem[b], add=True)
                if u + 2 < _G:
                    if u - 2 >= 0:
                        sdescs[u - 2].wait()
                    _issue_g(u + 2)
                else:
                    sdescs[u - 2].wait()
            for d in edescs:
                d.wait()
            return 0

        lax.fori_loop(0, _BLK, _block, 0)
        pltpu.make_async_copy(rows[2], acc.at[dbuf.at[_G - 2]], ssem[2]).wait()
        pltpu.make_async_copy(rows[3], acc.at[dbuf.at[_G - 1]], ssem[3]).wait()
        plsc.subcore_barrier()

        pltpu.sync_copy(acc.at[pl.ds(sid * npt, npt)],
                        acc_hbm.at[cid, pl.ds(sid * npt, npt)])
        pltpu.sync_copy(ews.at[pl.ds(sid * npt, npt)],
                        ews_hbm.at[cid, pl.ds(sid * npt, npt)])

    return sc_scatter


def kernel(x, edge_index, edge_width, vertex_mass):
    n = x.shape[0]
    e = edge_index.shape[1]
    src = edge_index[0].astype(jnp.int32)
    dst = edge_index[1].astype(jnp.int32)
    w = edge_width.reshape(e).astype(jnp.float32)

    wvec, freqs, amp = _probe_consts()
    probe = _make_probe(n)(x, wvec, freqs, amp)

    nw = 32
    epw = e // nw
    epw_pad = _CH * _G * _BLK
    pad = epw_pad - epw
    didx = jnp.broadcast_to(
        jnp.arange(pad, dtype=jnp.int32) % n, (nw, pad))
    src_p = jnp.concatenate(
        [src.reshape(nw, epw), didx], axis=1).reshape(-1, _CH)
    dst_p = jnp.concatenate(
        [dst.reshape(nw, epw), didx], axis=1).reshape(-1, _CH)
    w_p = jnp.concatenate(
        [w.reshape(nw, epw), jnp.zeros((nw, pad), jnp.float32)],
        axis=1).reshape(-1, _CH)

    accs, ews = _make_sc_scatter(n, epw_pad // _CH)(probe, src_p, dst_p, w_p)

    out = _make_combine(n)(
        accs[0, :n], accs[1, :n],
        ews[0, :n, None], ews[1, :n, None],
        probe, vertex_mass)
    return out
